# Initial kernel scaffold; baseline (speedup 1.0000x reference)
import functools

import numpy as np
import jax
import jax.numpy as jnp
from jax import lax
from jax.experimental import pallas as pl
from jax.experimental.pallas import tpu as pltpu
from jax.experimental.pallas import tpu_sc as plsc

NUM_LEVELS = 16
F = 2
T = 2 ** 19
MASK = T - 1
_growth = np.exp((np.log(256.0) - np.log(16.0)) / (NUM_LEVELS - 1))
_SCALINGS = np.floor(16.0 * _growth ** np.arange(NUM_LEVELS)).astype(np.float32)
_P = [1, -1640531535, 805459861, -620313867]

N = 65536
NW = 32
PW = N // NW
NGROUP = PW // 16


def _make_kernel():
    mesh = plsc.VectorSubcoreMesh(
        core_axis_name="c", subcore_axis_name="s", num_cores=2, num_subcores=16
    )

    @functools.partial(
        pl.kernel,
        out_type=jax.ShapeDtypeStruct((N, F * NUM_LEVELS), jnp.float32),
        mesh=mesh,
        scratch_types=[
            pltpu.VMEM((PW, 4), jnp.float32),
            pltpu.VMEM((16,), jnp.float32),
            pltpu.VMEM((32, 128), jnp.int32),
            pltpu.VMEM((32, 128), jnp.int32),
            pltpu.VMEM((256, 16), jnp.float32),
            pltpu.VMEM((256, 16), jnp.float32),
            pltpu.VMEM((32, 128, F), jnp.float32),
            pltpu.VMEM((32, 128, F), jnp.float32),
            pltpu.VMEM((PW, F * NUM_LEVELS), jnp.float32),
            pltpu.SemaphoreType.DMA,
            pltpu.SemaphoreType.DMA,
        ],
    )
    def encode(xyzt_hbm, table_hbm, scal_hbm, out_hbm,
               x_v, scal_v, idx_a, idx_b, w_a, w_b, rows_a, rows_b, out_v,
               sem_a, sem_b):
        cid = lax.axis_index("c")
        sid = lax.axis_index("s")
        wid = sid * 2 + cid
        base = wid * PW
        pltpu.sync_copy(xyzt_hbm.at[pl.ds(base, PW), :], x_v)
        pltpu.sync_copy(scal_hbm, scal_v)

        iota = lax.iota(jnp.int32, (16,))
        zeros16 = jnp.zeros((16,), jnp.int32)
        ones16 = jnp.full((16,), 1, jnp.int32)

        def phase_a(g, idx_ref, w_ref):
            rows = g * 16 + iota
            xg = [plsc.load_gather(x_v, [rows, jnp.full((16,), d, jnp.int32)])
                  for d in range(4)]

            @pl.loop(np.int32(0), np.int32(NUM_LEVELS))
            def _lvl(l):
                s = lax.broadcast(scal_v[l], (16,))
                lofs = lax.broadcast(l * T, (16,))
                m0, m1, off, om = [], [], [], []
                for d in range(4):
                    scaled = xg[d] * s
                    sf = jnp.floor(scaled)
                    off_d = scaled - sf
                    om_d = 1.0 - off_d
                    sfi = sf.astype(jnp.int32)
                    m0_d = sfi if d == 0 else sfi * _P[d]
                    m1_d = m0_d + _P[d]
                    m0.append(m0_d); m1.append(m1_d)
                    off.append(off_d); om.append(om_d)
                a01 = [m0[0] ^ m0[1], m1[0] ^ m0[1], m0[0] ^ m1[1], m1[0] ^ m1[1]]
                w01 = [om[0] * om[1], off[0] * om[1], om[0] * off[1], off[0] * off[1]]
                a23 = [m0[2] ^ m0[3], m1[2] ^ m0[3], m0[2] ^ m1[3], m1[2] ^ m1[3]]
                w23 = [om[2] * om[3], off[2] * om[3], om[2] * off[3], off[2] * off[3]]
                for c in range(16):
                    idxv = ((a01[c & 3] ^ a23[(c >> 2) & 3]) & MASK) + lofs
                    wv = w01[c & 3] * w23[(c >> 2) & 3]
                    r = l * 2 + (c >> 3)
                    idx_ref[r, pl.ds((c & 7) * 16, 16)] = idxv
                    w_ref[l * 16 + c, :] = wv

        def fire(idx_ref, rows_ref, sem):
            for k in range(32):
                pltpu.async_copy(table_hbm.at[idx_ref.at[k]], rows_ref.at[k], sem)

        def drain(idx_ref, rows_ref, sem):
            for k in range(32):
                pltpu.make_async_copy(
                    table_hbm.at[idx_ref.at[k]], rows_ref.at[k], sem
                ).wait()

        def phase_c(g, w_ref, rows_ref):
            orow = g * 16 + iota

            @pl.loop(np.int32(0), np.int32(NUM_LEVELS))
            def _lvl(l):
                acc0 = jnp.zeros((16,), jnp.float32)
                acc1 = jnp.zeros((16,), jnp.float32)
                for c in range(16):
                    rr = jnp.full((16,), l * 2 + (c >> 3), jnp.int32)
                    cols = (c & 7) * 16 + iota
                    f0 = plsc.load_gather(rows_ref, [rr, cols, zeros16])
                    f1 = plsc.load_gather(rows_ref, [rr, cols, ones16])
                    wv = w_ref[l * 16 + c, :]
                    acc0 = acc0 + wv * f0
                    acc1 = acc1 + wv * f1
                plsc.store_scatter(out_v, [orow, jnp.full((16,), 2 * l, jnp.int32)], acc0)
                plsc.store_scatter(out_v, [orow, jnp.full((16,), 2 * l + 1, jnp.int32)], acc1)

        phase_a(np.int32(0), idx_a, w_a)
        fire(idx_a, rows_a, sem_a)

        @pl.loop(np.int32(0), np.int32(NGROUP // 2 - 1))
        def _grp(k):
            g = k * 2
            phase_a(g + 1, idx_b, w_b)
            fire(idx_b, rows_b, sem_b)
            drain(idx_a, rows_a, sem_a)
            phase_c(g, w_a, rows_a)
            phase_a(g + 2, idx_a, w_a)
            fire(idx_a, rows_a, sem_a)
            drain(idx_b, rows_b, sem_b)
            phase_c(g + 1, w_b, rows_b)

        phase_a(np.int32(NGROUP - 1), idx_b, w_b)
        fire(idx_b, rows_b, sem_b)
        drain(idx_a, rows_a, sem_a)
        phase_c(np.int32(NGROUP - 2), w_a, rows_a)
        drain(idx_b, rows_b, sem_b)
        phase_c(np.int32(NGROUP - 1), w_b, rows_b)

        pltpu.sync_copy(out_v, out_hbm.at[pl.ds(base, PW), :])

    return encode


_encode = _make_kernel()


def kernel(xyzt, hash_table):
    scal = jnp.asarray(_SCALINGS)
    return _encode(xyzt.astype(jnp.float32), hash_table.astype(jnp.float32), scal)


# speedup vs baseline: 1.5091x; 1.5091x over previous
import functools

import numpy as np
import jax
import jax.numpy as jnp
from jax import lax
from jax.experimental import pallas as pl
from jax.experimental.pallas import tpu as pltpu
from jax.experimental.pallas import tpu_sc as plsc

NUM_LEVELS = 16
F = 2
T = 2 ** 19
MASK = T - 1
_growth = np.exp((np.log(256.0) - np.log(16.0)) / (NUM_LEVELS - 1))
_SCALINGS = np.floor(16.0 * _growth ** np.arange(NUM_LEVELS)).astype(np.float32)
_P = [1, -1640531535, 805459861, -620313867]

N = 65536
NW = 32
PW = N // NW
NGROUP = PW // 16


def _make_kernel():
    mesh = plsc.VectorSubcoreMesh(
        core_axis_name="c", subcore_axis_name="s", num_cores=2, num_subcores=16
    )

    @functools.partial(
        pl.kernel,
        out_type=jax.ShapeDtypeStruct((F * NUM_LEVELS, N), jnp.float32),
        mesh=mesh,
        scratch_types=[
            pltpu.VMEM((4, PW), jnp.float32),
            pltpu.VMEM((16, 16), jnp.float32),
            pltpu.VMEM((64, 128), jnp.int32),
            pltpu.VMEM((64, 128), jnp.int32),
            pltpu.VMEM((256, 16), jnp.float32),
            pltpu.VMEM((256, 16), jnp.float32),
            pltpu.VMEM((4096,), jnp.float32),
            pltpu.VMEM((4096,), jnp.float32),
            pltpu.VMEM((4096,), jnp.float32),
            pltpu.VMEM((4096,), jnp.float32),
            pltpu.VMEM((F * NUM_LEVELS, 256), jnp.float32),
            pltpu.SemaphoreType.DMA,
            pltpu.SemaphoreType.DMA,
        ],
    )
    def encode(xt_hbm, table_hbm, scal_hbm, out_hbm,
               x_v, scal_v, idx_a, idx_b, w_a, w_b,
               rows0_a, rows1_a, rows0_b, rows1_b, out_s,
               sem_a, sem_b):
        cid = lax.axis_index("c")
        sid = lax.axis_index("s")
        wid = sid * 2 + cid
        base = pl.multiple_of(wid * PW, PW)
        pltpu.sync_copy(xt_hbm.at[:, pl.ds(base, PW)], x_v)
        pltpu.sync_copy(scal_hbm, scal_v)

        def phase_a(g, idx_ref, w_ref):
            xg = [x_v[d, pl.ds(g * 16, 16)] for d in range(4)]

            @pl.loop(0, NUM_LEVELS)
            def _lvl(l):
                s = scal_v[l, :]
                lofs = lax.broadcast(l * T, (16,))
                m0, m1, off, om = [], [], [], []
                for d in range(4):
                    scaled = xg[d] * s
                    sfi = scaled.astype(jnp.int32)
                    sf = sfi.astype(jnp.float32)
                    off_d = scaled - sf
                    om_d = 1.0 - off_d
                    m0_d = sfi if d == 0 else sfi * _P[d]
                    m1_d = m0_d + _P[d]
                    m0.append(m0_d); m1.append(m1_d)
                    off.append(off_d); om.append(om_d)
                a01 = [m0[0] ^ m0[1], m1[0] ^ m0[1], m0[0] ^ m1[1], m1[0] ^ m1[1]]
                w01 = [om[0] * om[1], off[0] * om[1], om[0] * off[1], off[0] * off[1]]
                a23 = [m0[2] ^ m0[3], m1[2] ^ m0[3], m0[2] ^ m1[3], m1[2] ^ m1[3]]
                w23 = [om[2] * om[3], off[2] * om[3], om[2] * off[3], off[2] * off[3]]
                for c in range(16):
                    idxv = ((a01[c & 3] ^ a23[(c >> 2) & 3]) & MASK) + lofs
                    e0 = idxv + idxv
                    r = l * 2 + (c >> 3)
                    col = (c & 7) * 16
                    idx_ref[r, pl.ds(col, 16)] = e0
                    idx_ref[r + 32, pl.ds(col, 16)] = e0 + 1
                    w_ref[l * 16 + c, :] = w01[c & 3] * w23[(c >> 2) & 3]

        def fire(idx_ref, rows0, rows1, sem):
            for k in range(32):
                pltpu.async_copy(
                    table_hbm.at[idx_ref.at[k]], rows0.at[pl.ds(k * 128, 128)], sem)
                pltpu.async_copy(
                    table_hbm.at[idx_ref.at[k + 32]], rows1.at[pl.ds(k * 128, 128)], sem)

        def drain(idx_ref, rows0, rows1, sem):
            for k in range(32):
                pltpu.make_async_copy(
                    table_hbm.at[idx_ref.at[k]], rows0.at[pl.ds(k * 128, 128)], sem
                ).wait()
                pltpu.make_async_copy(
                    table_hbm.at[idx_ref.at[k + 32]], rows1.at[pl.ds(k * 128, 128)], sem
                ).wait()

        def phase_c(g, w_ref, rows0, rows1):
            gc = (g & 15) * 16

            @pl.loop(0, NUM_LEVELS)
            def _lvl(l):
                acc0 = jnp.zeros((16,), jnp.float32)
                acc1 = jnp.zeros((16,), jnp.float32)
                for c in range(16):
                    j = l * 16 + c
                    v0 = rows0[pl.ds(j * 16, 16)]
                    v1 = rows1[pl.ds(j * 16, 16)]
                    wv = w_ref[j, :]
                    acc0 = acc0 + wv * v0
                    acc1 = acc1 + wv * v1
                out_s[l * 2, pl.ds(gc, 16)] = acc0
                out_s[l * 2 + 1, pl.ds(gc, 16)] = acc1

            @pl.when((g & 15) == 15)
            def _flush():
                pltpu.sync_copy(
                    out_s,
                    out_hbm.at[:, pl.ds(pl.multiple_of(base + (g - 15) * 16, 256),
                                        256)])

        phase_a(0, idx_a, w_a)
        fire(idx_a, rows0_a, rows1_a, sem_a)

        @pl.loop(0, NGROUP // 2 - 1)
        def _grp(k):
            g = k * 2
            phase_a(g + 1, idx_b, w_b)
            fire(idx_b, rows0_b, rows1_b, sem_b)
            drain(idx_a, rows0_a, rows1_a, sem_a)
            phase_c(g, w_a, rows0_a, rows1_a)
            phase_a(g + 2, idx_a, w_a)
            fire(idx_a, rows0_a, rows1_a, sem_a)
            drain(idx_b, rows0_b, rows1_b, sem_b)
            phase_c(g + 1, w_b, rows0_b, rows1_b)

        phase_a(NGROUP - 1, idx_b, w_b)
        fire(idx_b, rows0_b, rows1_b, sem_b)
        drain(idx_a, rows0_a, rows1_a, sem_a)
        phase_c(NGROUP - 2, w_a, rows0_a, rows1_a)
        drain(idx_b, rows0_b, rows1_b, sem_b)
        phase_c(NGROUP - 1, w_b, rows0_b, rows1_b)

    return encode


_encode = _make_kernel()


def kernel(xyzt, hash_table):
    with jax.enable_x64(False):
        xt = xyzt.astype(jnp.float32).T
        table_flat = hash_table.astype(jnp.float32).reshape(-1)
        scal = jnp.broadcast_to(jnp.asarray(_SCALINGS)[:, None], (16, 16))
        out3 = _encode(xt, table_flat, scal)
        return out3.T
